# Initial kernel scaffold; baseline (speedup 1.0000x reference)
import jax
import jax.numpy as jnp
from jax.experimental import pallas as pl


def kernel(node_states_scalar, node_states_vector, edge_states, edge_vectors, edge_norms, edge_index, Wf, bf, Ws1, bs1, Ws2, bs2, U, V, Wa1, ba1, Wa2, ba2):
    raise NotImplementedError("write your pallas kernel here")


# speedup vs baseline: 6.5263x; 6.5263x over previous
import functools

import jax
import jax.numpy as jnp
from jax import lax
from jax.experimental import pallas as pl
from jax.experimental.pallas import tpu as pltpu
from jax.experimental.pallas import tpu_sc as plsc

N = 10000
E = 320000
D = 128
CUTOFF = 5.0

NC = 2
NS = 16
NW = NC * NS
EPW = E // NW
BE = 40
NBLK = EPW // BE
NPAD = 10112
RPT = NPAD // NS
NCH = D // 16


def _silu(x):
    return x * (1.0 / (1.0 + jnp.exp(-x)))


_EB = 2000


def _edge_filter_body(es_ref, nrm_ref, ev_ref, wf_ref, bf_ref,
                      fn_ref, fs_ref, fex_ref, fey_ref, fez_ref):
    x = jnp.dot(es_ref[...], wf_ref[...], preferred_element_type=jnp.float32)
    x = x + bf_ref[...]
    r = nrm_ref[...]
    cut = jnp.where(r < CUTOFF, 0.5 * (jnp.cos(jnp.pi * r / CUTOFF) + 1.0), 0.0)
    x = x * cut
    fn_ref[...] = x[:, :D]
    fe = x[:, D:2 * D]
    fs_ref[...] = x[:, 2 * D:]
    ev = ev_ref[...]
    fex_ref[...] = fe * ev[:, 0:1]
    fey_ref[...] = fe * ev[:, 1:2]
    fez_ref[...] = fe * ev[:, 2:3]


def _edge_filter(edge_states, edge_norms, edge_vectors, Wf, bf):
    grid = (E // _EB,)
    return pl.pallas_call(
        _edge_filter_body,
        grid=grid,
        in_specs=[
            pl.BlockSpec((_EB, 16), lambda i: (i, 0)),
            pl.BlockSpec((_EB, 1), lambda i: (i, 0)),
            pl.BlockSpec((_EB, 3), lambda i: (i, 0)),
            pl.BlockSpec((16, 3 * D), lambda i: (0, 0)),
            pl.BlockSpec((1, 3 * D), lambda i: (0, 0)),
        ],
        out_specs=[pl.BlockSpec((_EB, D), lambda i: (i, 0))] * 5,
        out_shape=[jax.ShapeDtypeStruct((E, D), jnp.float32)] * 5,
    )(edge_states, edge_norms, edge_vectors, Wf, bf.reshape(1, 3 * D))


_NB = 2000


def _node_mlp_body(s_ref, w1_ref, b1_ref, w2_ref, b2_ref, on_ref, oe_ref, os_ref):
    h = jnp.dot(s_ref[...], w1_ref[...], preferred_element_type=jnp.float32)
    h = _silu(h + b1_ref[...])
    y = jnp.dot(h, w2_ref[...], preferred_element_type=jnp.float32) + b2_ref[...]
    on_ref[...] = y[:, :D]
    oe_ref[...] = y[:, D:2 * D]
    os_ref[...] = y[:, 2 * D:]


def _node_mlp(node_states_scalar, Ws1, bs1, Ws2, bs2):
    grid = (N // _NB,)
    return pl.pallas_call(
        _node_mlp_body,
        grid=grid,
        in_specs=[
            pl.BlockSpec((_NB, D), lambda i: (i, 0)),
            pl.BlockSpec((D, D), lambda i: (0, 0)),
            pl.BlockSpec((1, D), lambda i: (0, 0)),
            pl.BlockSpec((D, 3 * D), lambda i: (0, 0)),
            pl.BlockSpec((1, 3 * D), lambda i: (0, 0)),
        ],
        out_specs=[pl.BlockSpec((_NB, D), lambda i: (i, 0))] * 3,
        out_shape=[jax.ShapeDtypeStruct((N, D), jnp.float32)] * 3,
    )(node_states_scalar, Ws1, bs1.reshape(1, D), Ws2, bs2.reshape(1, 3 * D))


def _sc_body(src_hbm, dst_hbm, zeros_hbm,
             fwn_hbm, fws_hbm,
             fex_hbm, fey_hbm, fez_hbm,
             son_hbm, soe_hbm, sos_hbm,
             nvx_hbm, nvy_hbm, nvz_hbm,
             os_hbm, ox_hbm, oy_hbm, oz_hbm,
             sidx, didx, b0, b1, b2, b3, b4, mb, acc, sem):
    cid = lax.axis_index("c")
    sid = lax.axis_index("s")
    wid = sid * NC + cid
    ebase = wid * EPW
    rbase = pl.multiple_of(sid * RPT, 8)

    def run_pass(scalar_pass, fwa_hbm, fwb_hbm, soa_hbm, sob_hbm, nv_hbm,
                 out_hbm):
        pltpu.sync_copy(zeros_hbm.at[pl.ds(rbase, RPT)], acc.at[pl.ds(rbase, RPT)])
        plsc.subcore_barrier()

        def blk(b, carry):
            off = pl.multiple_of(ebase + b * BE, 8)
            pltpu.sync_copy(src_hbm.at[pl.ds(off, BE)], sidx)
            pltpu.sync_copy(dst_hbm.at[pl.ds(off, BE)], didx)
            if scalar_pass:
                pltpu.sync_copy(fwa_hbm.at[pl.ds(off, BE)], b0)
                pltpu.async_copy(soa_hbm.at[sidx], b2, sem).wait()

                def edge(i, c):
                    for j in range(NCH):
                        sl = pl.ds(j * 16, 16)
                        mb[i, sl] = b0[i, sl] * b2[i, sl]
                    return c

                lax.fori_loop(0, BE, edge, 0)
            else:
                pltpu.sync_copy(fwa_hbm.at[pl.ds(off, BE)], b0)
                pltpu.sync_copy(fwb_hbm.at[pl.ds(off, BE)], b1)
                pltpu.async_copy(soa_hbm.at[sidx], b2, sem).wait()
                pltpu.async_copy(sob_hbm.at[sidx], b3, sem).wait()
                pltpu.async_copy(nv_hbm.at[sidx], b4, sem).wait()

                def edge(i, c):
                    for j in range(NCH):
                        sl = pl.ds(j * 16, 16)
                        mb[i, sl] = (b0[i, sl] * b2[i, sl] * b4[i, sl]
                                     + b1[i, sl] * b3[i, sl])
                    return c

                lax.fori_loop(0, BE, edge, 0)
            pltpu.sync_copy(mb, acc.at[didx], add=True)
            return carry

        lax.fori_loop(0, NBLK, blk, 0)
        plsc.subcore_barrier()
        pltpu.sync_copy(acc.at[pl.ds(rbase, RPT)],
                        out_hbm.at[cid].at[pl.ds(rbase, RPT)])
        plsc.subcore_barrier()

    run_pass(True, fws_hbm, None, sos_hbm, None, None, os_hbm)
    run_pass(False, fwn_hbm, fex_hbm, son_hbm, soe_hbm, nvx_hbm, ox_hbm)
    run_pass(False, fwn_hbm, fey_hbm, son_hbm, soe_hbm, nvy_hbm, oy_hbm)
    run_pass(False, fwn_hbm, fez_hbm, son_hbm, soe_hbm, nvz_hbm, oz_hbm)


def _sc_scatter(src, dst, zeros_nd, fwn, fws, fex, fey, fez, son, soe, sos,
                nvx, nvy, nvz):
    mesh = plsc.VectorSubcoreMesh(core_axis_name="c", subcore_axis_name="s",
                                  num_cores=NC, num_subcores=NS)
    f = pl.kernel(
        _sc_body,
        out_type=[jax.ShapeDtypeStruct((NC, NPAD, D), jnp.float32)] * 4,
        mesh=mesh,
        scratch_types=[
            pltpu.VMEM((BE,), jnp.int32),
            pltpu.VMEM((BE,), jnp.int32),
            pltpu.VMEM((BE, D), jnp.float32),
            pltpu.VMEM((BE, D), jnp.float32),
            pltpu.VMEM((BE, D), jnp.float32),
            pltpu.VMEM((BE, D), jnp.float32),
            pltpu.VMEM((BE, D), jnp.float32),
            pltpu.VMEM((BE, D), jnp.float32),
            pltpu.VMEM_SHARED((NPAD, D), jnp.float32),
            pltpu.SemaphoreType.DMA,
        ],
    )
    return f(src, dst, zeros_nd, fwn, fws, fex, fey, fez, son, soe, sos,
             nvx, nvy, nvz)


def _update_body(s_ref, vx_ref, vy_ref, vz_ref,
                 ps_ref, px_ref, py_ref, pz_ref,
                 u_ref, v_ref, wa1_ref, ba1_ref, wa2_ref, ba2_ref,
                 os_ref, ovx_ref, ovy_ref, ovz_ref):
    ns = s_ref[...] + ps_ref[0] + ps_ref[1]
    nvx = vx_ref[...] + px_ref[0] + px_ref[1]
    nvy = vy_ref[...] + py_ref[0] + py_ref[1]
    nvz = vz_ref[...] + pz_ref[0] + pz_ref[1]
    U = u_ref[...]
    V = v_ref[...]
    ux = jnp.dot(nvx, U, preferred_element_type=jnp.float32)
    uy = jnp.dot(nvy, U, preferred_element_type=jnp.float32)
    uz = jnp.dot(nvz, U, preferred_element_type=jnp.float32)
    wx = jnp.dot(nvx, V, preferred_element_type=jnp.float32)
    wy = jnp.dot(nvy, V, preferred_element_type=jnp.float32)
    wz = jnp.dot(nvz, V, preferred_element_type=jnp.float32)
    vv_sq = wx * wx + wy * wy + wz * wz
    h = jnp.concatenate((ns, vv_sq), axis=1)
    h = jnp.dot(h, wa1_ref[...], preferred_element_type=jnp.float32)
    h = _silu(h + ba1_ref[...])
    a = jnp.dot(h, wa2_ref[...], preferred_element_type=jnp.float32) + ba2_ref[...]
    a_ss = a[:, :D]
    a_sv = a[:, D:2 * D]
    a_vv = a[:, 2 * D:]
    inner = ux * wx + uy * wy + uz * wz
    os_ref[...] = ns + a_ss + a_sv * inner
    ovx_ref[...] = nvx + a_vv * ux
    ovy_ref[...] = nvy + a_vv * uy
    ovz_ref[...] = nvz + a_vv * uz


def _node_update(s, vx, vy, vz, ps, px, py, pz, U, V, Wa1, ba1, Wa2, ba2):
    grid = (N // _NB,)
    nspec = pl.BlockSpec((_NB, D), lambda i: (i, 0))
    pspec = pl.BlockSpec((NC, _NB, D), lambda i: (0, i, 0))
    return pl.pallas_call(
        _update_body,
        grid=grid,
        in_specs=[
            nspec, nspec, nspec, nspec,
            pspec, pspec, pspec, pspec,
            pl.BlockSpec((D, D), lambda i: (0, 0)),
            pl.BlockSpec((D, D), lambda i: (0, 0)),
            pl.BlockSpec((2 * D, D), lambda i: (0, 0)),
            pl.BlockSpec((1, D), lambda i: (0, 0)),
            pl.BlockSpec((D, 3 * D), lambda i: (0, 0)),
            pl.BlockSpec((1, 3 * D), lambda i: (0, 0)),
        ],
        out_specs=[nspec] * 4,
        out_shape=[jax.ShapeDtypeStruct((N, D), jnp.float32)] * 4,
    )(s, vx, vy, vz, ps, px, py, pz, U, V, Wa1, ba1.reshape(1, D),
      Wa2, ba2.reshape(1, 3 * D))


@jax.jit
def kernel(node_states_scalar, node_states_vector, edge_states, edge_vectors,
           edge_norms, edge_index, Wf, bf, Ws1, bs1, Ws2, bs2, U, V,
           Wa1, ba1, Wa2, ba2):
    fwn, fws, fex, fey, fez = _edge_filter(edge_states, edge_norms,
                                           edge_vectors, Wf, bf)
    son, soe, sos = _node_mlp(node_states_scalar, Ws1, bs1, Ws2, bs2)

    src = edge_index[:, 0]
    dst = edge_index[:, 1]
    nvx = node_states_vector[:, 0, :]
    nvy = node_states_vector[:, 1, :]
    nvz = node_states_vector[:, 2, :]
    zeros_nd = jnp.zeros((NPAD, D), jnp.float32)

    ps, px, py, pz = _sc_scatter(src, dst, zeros_nd, fwn, fws, fex, fey, fez,
                                 son, soe, sos, nvx, nvy, nvz)

    os_, ovx, ovy, ovz = _node_update(node_states_scalar, nvx, nvy, nvz,
                                      ps, px, py, pz, U, V, Wa1, ba1, Wa2, ba2)
    return os_, jnp.stack((ovx, ovy, ovz), axis=1)


# speedup vs baseline: 10.9791x; 1.6823x over previous
import functools

import jax
import jax.numpy as jnp
from jax import lax
from jax.experimental import pallas as pl
from jax.experimental.pallas import tpu as pltpu
from jax.experimental.pallas import tpu_sc as plsc

N = 10000
E = 320000
D = 128
CUTOFF = 5.0

NC = 2
NS = 16
NW = NC * NS
EPW = E // NW
BE = 40
NBLK = EPW // BE
NPAD = 10112
RPT = NPAD // NS
NCH = D // 16


def _silu(x):
    return x * (1.0 / (1.0 + jnp.exp(-x)))


_EB = 2000


def _edge_filter_body(es_ref, nrm_ref, ev_ref, wf_ref, bf_ref,
                      fn_ref, fs_ref, fex_ref, fey_ref, fez_ref):
    x = jnp.dot(es_ref[...], wf_ref[...], preferred_element_type=jnp.float32)
    x = x + bf_ref[...]
    r = nrm_ref[...]
    cut = jnp.where(r < CUTOFF, 0.5 * (jnp.cos(jnp.pi * r / CUTOFF) + 1.0), 0.0)
    x = x * cut
    fn_ref[...] = x[:, :D]
    fe = x[:, D:2 * D]
    fs_ref[...] = x[:, 2 * D:]
    ev = ev_ref[...]
    fex_ref[...] = fe * ev[:, 0:1]
    fey_ref[...] = fe * ev[:, 1:2]
    fez_ref[...] = fe * ev[:, 2:3]


def _edge_filter(edge_states, edge_norms, edge_vectors, Wf, bf):
    grid = (E // _EB,)
    return pl.pallas_call(
        _edge_filter_body,
        grid=grid,
        in_specs=[
            pl.BlockSpec((_EB, 16), lambda i: (i, 0)),
            pl.BlockSpec((_EB, 1), lambda i: (i, 0)),
            pl.BlockSpec((_EB, 3), lambda i: (i, 0)),
            pl.BlockSpec((16, 3 * D), lambda i: (0, 0)),
            pl.BlockSpec((1, 3 * D), lambda i: (0, 0)),
        ],
        out_specs=[pl.BlockSpec((_EB, D), lambda i: (i, 0))] * 5,
        out_shape=[jax.ShapeDtypeStruct((E, D), jnp.float32)] * 5,
    )(edge_states, edge_norms, edge_vectors, Wf, bf.reshape(1, 3 * D))


_NB = 2000


def _node_mlp_body(s_ref, w1_ref, b1_ref, w2_ref, b2_ref, on_ref, oe_ref, os_ref):
    h = jnp.dot(s_ref[...], w1_ref[...], preferred_element_type=jnp.float32)
    h = _silu(h + b1_ref[...])
    y = jnp.dot(h, w2_ref[...], preferred_element_type=jnp.float32) + b2_ref[...]
    on_ref[...] = y[:, :D]
    oe_ref[...] = y[:, D:2 * D]
    os_ref[...] = y[:, 2 * D:]


def _node_mlp(node_states_scalar, Ws1, bs1, Ws2, bs2):
    grid = (N // _NB,)
    return pl.pallas_call(
        _node_mlp_body,
        grid=grid,
        in_specs=[
            pl.BlockSpec((_NB, D), lambda i: (i, 0)),
            pl.BlockSpec((D, D), lambda i: (0, 0)),
            pl.BlockSpec((1, D), lambda i: (0, 0)),
            pl.BlockSpec((D, 3 * D), lambda i: (0, 0)),
            pl.BlockSpec((1, 3 * D), lambda i: (0, 0)),
        ],
        out_specs=[pl.BlockSpec((_NB, D), lambda i: (i, 0))] * 3,
        out_shape=[jax.ShapeDtypeStruct((N, D), jnp.float32)] * 3,
    )(node_states_scalar, Ws1, bs1.reshape(1, D), Ws2, bs2.reshape(1, 3 * D))


def _sc_body(src_hbm, dst_hbm, zeros_hbm,
             fwn_hbm, fws_hbm,
             fex_hbm, fey_hbm, fez_hbm,
             son_hbm, soe_hbm, sos_hbm,
             nvx_hbm, nvy_hbm, nvz_hbm,
             os_hbm, ox_hbm, oy_hbm, oz_hbm,
             sidxA, sidxB, didx, b0, b1,
             g2A, g3A, g4A, g2B, g3B, g4B,
             acc, semgA, semgB, semf):
    cid = lax.axis_index("c")
    sid = lax.axis_index("s")
    wid = sid * NC + cid
    ebase = wid * EPW
    rbase = pl.multiple_of(sid * RPT, 8)

    def run_pass(scalar_pass, fwa_hbm, fwb_hbm, soa_hbm, sob_hbm, nv_hbm,
                 out_hbm):
        pltpu.sync_copy(zeros_hbm.at[pl.ds(rbase, RPT)], acc.at[pl.ds(rbase, RPT)])
        plsc.subcore_barrier()

        def issue(b, sidx, g2, g3, g4, semg):
            off = pl.multiple_of(ebase + b * BE, 8)
            pltpu.sync_copy(src_hbm.at[pl.ds(off, BE)], sidx)
            pltpu.async_copy(soa_hbm.at[sidx], g2, semg)
            if not scalar_pass:
                pltpu.async_copy(sob_hbm.at[sidx], g3, semg)
                pltpu.async_copy(nv_hbm.at[sidx], g4, semg)

        def work(b, sidx, g2, g3, g4, semg):
            off = pl.multiple_of(ebase + b * BE, 8)
            pltpu.async_copy(fwa_hbm.at[pl.ds(off, BE)], b0, semf)
            if not scalar_pass:
                pltpu.async_copy(fwb_hbm.at[pl.ds(off, BE)], b1, semf)
            pltpu.async_copy(dst_hbm.at[pl.ds(off, BE)], didx, semf)
            pltpu.make_async_copy(soa_hbm.at[sidx], g2, semg).wait()
            if not scalar_pass:
                pltpu.make_async_copy(sob_hbm.at[sidx], g3, semg).wait()
                pltpu.make_async_copy(nv_hbm.at[sidx], g4, semg).wait()
            pltpu.make_async_copy(fwa_hbm.at[pl.ds(off, BE)], b0, semf).wait()
            if not scalar_pass:
                pltpu.make_async_copy(fwb_hbm.at[pl.ds(off, BE)], b1, semf).wait()
            pltpu.make_async_copy(dst_hbm.at[pl.ds(off, BE)], didx, semf).wait()
            if scalar_pass:
                def edge(i, c):
                    for j in range(NCH):
                        sl = pl.ds(j * 16, 16)
                        g2[i, sl] = b0[i, sl] * g2[i, sl]
                    return c

                lax.fori_loop(0, BE, edge, 0)
                mb = g2
            else:
                def edge(i, c):
                    for j in range(NCH):
                        sl = pl.ds(j * 16, 16)
                        g3[i, sl] = (b1[i, sl] * g3[i, sl]
                                     + b0[i, sl] * g2[i, sl] * g4[i, sl])
                    return c

                lax.fori_loop(0, BE, edge, 0)
                mb = g3
            pltpu.sync_copy(mb, acc.at[didx], add=True)

        issue(0, sidxA, g2A, g3A, g4A, semgA)

        def pair(g, carry):
            issue(2 * g + 1, sidxB, g2B, g3B, g4B, semgB)
            work(2 * g, sidxA, g2A, g3A, g4A, semgA)

            @pl.when(2 * g + 2 < NBLK)
            def _():
                issue(2 * g + 2, sidxA, g2A, g3A, g4A, semgA)

            work(2 * g + 1, sidxB, g2B, g3B, g4B, semgB)
            return carry

        lax.fori_loop(0, NBLK // 2, pair, 0)
        plsc.subcore_barrier()
        pltpu.sync_copy(acc.at[pl.ds(rbase, RPT)],
                        out_hbm.at[cid].at[pl.ds(rbase, RPT)])
        plsc.subcore_barrier()

    run_pass(True, fws_hbm, None, sos_hbm, None, None, os_hbm)
    run_pass(False, fwn_hbm, fex_hbm, son_hbm, soe_hbm, nvx_hbm, ox_hbm)
    run_pass(False, fwn_hbm, fey_hbm, son_hbm, soe_hbm, nvy_hbm, oy_hbm)
    run_pass(False, fwn_hbm, fez_hbm, son_hbm, soe_hbm, nvz_hbm, oz_hbm)


def _sc_scatter(src, dst, zeros_nd, fwn, fws, fex, fey, fez, son, soe, sos,
                nvx, nvy, nvz):
    mesh = plsc.VectorSubcoreMesh(core_axis_name="c", subcore_axis_name="s",
                                  num_cores=NC, num_subcores=NS)
    f = pl.kernel(
        _sc_body,
        out_type=[jax.ShapeDtypeStruct((NC, NPAD, D), jnp.float32)] * 4,
        mesh=mesh,
        scratch_types=[
            pltpu.VMEM((BE,), jnp.int32),
            pltpu.VMEM((BE,), jnp.int32),
            pltpu.VMEM((BE,), jnp.int32),
            pltpu.VMEM((BE, D), jnp.float32),
            pltpu.VMEM((BE, D), jnp.float32),
            pltpu.VMEM((BE, D), jnp.float32),
            pltpu.VMEM((BE, D), jnp.float32),
            pltpu.VMEM((BE, D), jnp.float32),
            pltpu.VMEM((BE, D), jnp.float32),
            pltpu.VMEM((BE, D), jnp.float32),
            pltpu.VMEM((BE, D), jnp.float32),
            pltpu.VMEM_SHARED((NPAD, D), jnp.float32),
            pltpu.SemaphoreType.DMA,
            pltpu.SemaphoreType.DMA,
            pltpu.SemaphoreType.DMA,
        ],
    )
    return f(src, dst, zeros_nd, fwn, fws, fex, fey, fez, son, soe, sos,
             nvx, nvy, nvz)


def _update_body(s_ref, vx_ref, vy_ref, vz_ref,
                 ps_ref, px_ref, py_ref, pz_ref,
                 u_ref, v_ref, wa1_ref, ba1_ref, wa2_ref, ba2_ref,
                 os_ref, ovx_ref, ovy_ref, ovz_ref):
    ns = s_ref[...] + ps_ref[0] + ps_ref[1]
    nvx = vx_ref[...] + px_ref[0] + px_ref[1]
    nvy = vy_ref[...] + py_ref[0] + py_ref[1]
    nvz = vz_ref[...] + pz_ref[0] + pz_ref[1]
    U = u_ref[...]
    V = v_ref[...]
    ux = jnp.dot(nvx, U, preferred_element_type=jnp.float32)
    uy = jnp.dot(nvy, U, preferred_element_type=jnp.float32)
    uz = jnp.dot(nvz, U, preferred_element_type=jnp.float32)
    wx = jnp.dot(nvx, V, preferred_element_type=jnp.float32)
    wy = jnp.dot(nvy, V, preferred_element_type=jnp.float32)
    wz = jnp.dot(nvz, V, preferred_element_type=jnp.float32)
    vv_sq = wx * wx + wy * wy + wz * wz
    h = jnp.concatenate((ns, vv_sq), axis=1)
    h = jnp.dot(h, wa1_ref[...], preferred_element_type=jnp.float32)
    h = _silu(h + ba1_ref[...])
    a = jnp.dot(h, wa2_ref[...], preferred_element_type=jnp.float32) + ba2_ref[...]
    a_ss = a[:, :D]
    a_sv = a[:, D:2 * D]
    a_vv = a[:, 2 * D:]
    inner = ux * wx + uy * wy + uz * wz
    os_ref[...] = ns + a_ss + a_sv * inner
    ovx_ref[...] = nvx + a_vv * ux
    ovy_ref[...] = nvy + a_vv * uy
    ovz_ref[...] = nvz + a_vv * uz


def _node_update(s, vx, vy, vz, ps, px, py, pz, U, V, Wa1, ba1, Wa2, ba2):
    grid = (N // _NB,)
    nspec = pl.BlockSpec((_NB, D), lambda i: (i, 0))
    pspec = pl.BlockSpec((NC, _NB, D), lambda i: (0, i, 0))
    return pl.pallas_call(
        _update_body,
        grid=grid,
        in_specs=[
            nspec, nspec, nspec, nspec,
            pspec, pspec, pspec, pspec,
            pl.BlockSpec((D, D), lambda i: (0, 0)),
            pl.BlockSpec((D, D), lambda i: (0, 0)),
            pl.BlockSpec((2 * D, D), lambda i: (0, 0)),
            pl.BlockSpec((1, D), lambda i: (0, 0)),
            pl.BlockSpec((D, 3 * D), lambda i: (0, 0)),
            pl.BlockSpec((1, 3 * D), lambda i: (0, 0)),
        ],
        out_specs=[nspec] * 4,
        out_shape=[jax.ShapeDtypeStruct((N, D), jnp.float32)] * 4,
    )(s, vx, vy, vz, ps, px, py, pz, U, V, Wa1, ba1.reshape(1, D),
      Wa2, ba2.reshape(1, 3 * D))


@jax.jit
def kernel(node_states_scalar, node_states_vector, edge_states, edge_vectors,
           edge_norms, edge_index, Wf, bf, Ws1, bs1, Ws2, bs2, U, V,
           Wa1, ba1, Wa2, ba2):
    fwn, fws, fex, fey, fez = _edge_filter(edge_states, edge_norms,
                                           edge_vectors, Wf, bf)
    son, soe, sos = _node_mlp(node_states_scalar, Ws1, bs1, Ws2, bs2)

    src = edge_index[:, 0]
    dst = edge_index[:, 1]
    nvx = node_states_vector[:, 0, :]
    nvy = node_states_vector[:, 1, :]
    nvz = node_states_vector[:, 2, :]
    zeros_nd = jnp.zeros((NPAD, D), jnp.float32)

    ps, px, py, pz = _sc_scatter(src, dst, zeros_nd, fwn, fws, fex, fey, fez,
                                 son, soe, sos, nvx, nvy, nvz)

    os_, ovx, ovy, ovz = _node_update(node_states_scalar, nvx, nvy, nvz,
                                      ps, px, py, pz, U, V, Wa1, ba1, Wa2, ba2)
    return os_, jnp.stack((ovx, ovy, ovz), axis=1)
